# baseline (speedup 1.0000x reference)
import functools

import jax
import jax.numpy as jnp
from jax import lax
from jax.experimental import pallas as pl
from jax.experimental.pallas import tpu as pltpu
from jax.experimental.pallas import tpu_sc as plsc

B, F, D = 128, 16, 1000
L = 16
NC, NS = 2, 16
NW = NC * NS
TP = 32

_mesh = plsc.VectorSubcoreMesh(core_axis_name="c", subcore_axis_name="s")


@functools.partial(
    pl.kernel,
    mesh=_mesh,
    out_type=jax.ShapeDtypeStruct((D, B), jnp.float32),
    scratch_types=[
        pltpu.VMEM((F, TP, B), jnp.float32),
        pltpu.VMEM((TP, B), jnp.float32),
        pltpu.SemaphoreType.DMA,
        pltpu.SemaphoreType.DMA,
    ],
)
def _mean_sc(x_hbm, out_hbm, x_v, o_v, s0, s1):
    wid = lax.axis_index("s") * NC + lax.axis_index("c")
    tile = jnp.minimum(wid * (TP // 8), D // 8 - TP // 8)
    start = pl.multiple_of(tile * 8, 8)
    copies = [
        pltpu.async_copy(
            x_hbm.at[f, pl.ds(start, TP), :], x_v.at[f], s0 if f < 8 else s1
        )
        for f in range(F)
    ]
    for cp in copies[:8]:
        cp.wait()

    def row1(r, carry):
        for c in range(B // L):
            sl = pl.ds(c * L, L)
            acc0 = x_v[0, r, sl] + x_v[1, r, sl]
            acc1 = x_v[2, r, sl] + x_v[3, r, sl]
            acc2 = x_v[4, r, sl] + x_v[5, r, sl]
            acc3 = x_v[6, r, sl] + x_v[7, r, sl]
            o_v[r, sl] = (acc0 + acc1) + (acc2 + acc3)
        return carry

    lax.fori_loop(0, TP, row1, 0)
    for cp in copies[8:]:
        cp.wait()

    def row2(r, carry):
        for c in range(B // L):
            sl = pl.ds(c * L, L)
            acc0 = o_v[r, sl] + x_v[8, r, sl]
            acc1 = x_v[9, r, sl] + x_v[10, r, sl]
            acc2 = x_v[11, r, sl] + x_v[12, r, sl]
            acc3 = x_v[13, r, sl] + x_v[14, r, sl]
            acc0 = acc0 + x_v[15, r, sl]
            o_v[r, sl] = ((acc0 + acc1) + (acc2 + acc3)) * (1.0 / F)
        return carry

    lax.fori_loop(0, TP, row2, 0)
    pltpu.sync_copy(o_v, out_hbm.at[pl.ds(start, TP), :])


def kernel(input):
    x_t = jnp.transpose(input, (1, 2, 0))
    out_t = _mean_sc(x_t)
    return jnp.transpose(out_t)[:, None, :]


# speedup vs baseline: 1.0148x; 1.0148x over previous
import functools

import jax
import jax.numpy as jnp
from jax import lax
from jax.experimental import pallas as pl
from jax.experimental.pallas import tpu as pltpu
from jax.experimental.pallas import tpu_sc as plsc

B, F, D = 128, 16, 1000
L = 16
NC, NS = 2, 16
NW = NC * NS
TP = 32

_mesh = plsc.VectorSubcoreMesh(core_axis_name="c", subcore_axis_name="s")


@functools.partial(
    pl.kernel,
    mesh=_mesh,
    out_type=jax.ShapeDtypeStruct((D, B), jnp.float32),
    scratch_types=[
        pltpu.VMEM((F, TP, B), jnp.float32),
        pltpu.VMEM((TP, B), jnp.float32),
        pltpu.SemaphoreType.DMA,
    ],
)
def _mean_sc(x_hbm, out_hbm, x_v, o_v, sem):
    wid = lax.axis_index("s") * NC + lax.axis_index("c")
    tile = jnp.minimum(wid * (TP // 8), D // 8 - TP // 8)
    start = pl.multiple_of(tile * 8, 8)
    copies = [
        pltpu.async_copy(x_hbm.at[f, pl.ds(start, TP), :], x_v.at[f], sem)
        for f in range(F)
    ]
    for cp in copies:
        cp.wait()

    def row(r, carry):
        for c in range(B // L):
            sl = pl.ds(c * L, L)
            acc0 = x_v[0, r, sl] + x_v[1, r, sl]
            acc1 = x_v[2, r, sl] + x_v[3, r, sl]
            acc2 = x_v[4, r, sl] + x_v[5, r, sl]
            acc3 = x_v[6, r, sl] + x_v[7, r, sl]
            for f in range(8, F):
                acc0 = acc0 + x_v[f, r, sl]
            o_v[r, sl] = ((acc0 + acc1) + (acc2 + acc3)) * (1.0 / F)
        return carry

    lax.fori_loop(0, TP, row, 0)
    pltpu.sync_copy(o_v, out_hbm.at[pl.ds(start, TP), :])


def kernel(input):
    x_t = jnp.transpose(input, (1, 2, 0))
    out_t = _mean_sc(x_t)
    return jnp.transpose(out_t)[:, None, :]
